# baseline (speedup 1.0000x reference)
import functools

import jax
import jax.numpy as jnp
from jax import lax
from jax.experimental import pallas as pl
from jax.experimental.pallas import tpu as pltpu
from jax.experimental.pallas import tpu_sc as plsc

NUM_FIELDS = 26
VOCAB = 100000
EMB = 32
B = 16384
H0 = 256
H1 = 128

LANES = 16
NC = 2
NS = 16
NW = NC * NS
BPW = B // NW
CHUNK = 8
NCHUNK = BPW // CHUNK
GROUPS = 2
IDX_PER_GROUP = (CHUNK // GROUPS) * NUM_FIELDS


def _sc_body(flat_idx_hbm, w2_hbm, fm2_hbm, idx_v, rows_v, out_v, sem):
    wid = lax.axis_index("c") * NS + lax.axis_index("s")
    row0 = wid * BPW

    def chunk_body(i, carry):
        base_row = row0 + i * CHUNK
        ibase = base_row * NUM_FIELDS
        for g in range(GROUPS):
            pltpu.sync_copy(
                flat_idx_hbm.at[pl.ds(ibase + g * IDX_PER_GROUP, IDX_PER_GROUP)],
                idx_v.at[g],
            )
        copies = [
            pltpu.async_copy(w2_hbm.at[idx_v.at[g]], rows_v.at[g], sem)
            for g in range(GROUPS)
        ]
        for c in copies:
            c.wait()
        rows_per_group = CHUNK // GROUPS
        for r in range(CHUNK):
            g = r // rows_per_group
            lo = (r % rows_per_group) * NUM_FIELDS
            acc0 = jnp.zeros((LANES,), jnp.float32)
            acc1 = jnp.zeros((LANES,), jnp.float32)
            sq0 = jnp.zeros((LANES,), jnp.float32)
            sq1 = jnp.zeros((LANES,), jnp.float32)
            for j in range(NUM_FIELDS):
                x0 = rows_v[g, lo + j, pl.ds(0, LANES)]
                x1 = rows_v[g, lo + j, pl.ds(LANES, LANES)]
                acc0 = acc0 + x0
                sq0 = sq0 + x0 * x0
                acc1 = acc1 + x1
                sq1 = sq1 + x1 * x1
            out_v[r, pl.ds(0, LANES)] = (acc0 * acc0 - sq0) * 0.005
            out_v[r, pl.ds(LANES, LANES)] = (acc1 * acc1 - sq1) * 0.005
        pltpu.sync_copy(out_v, fm2_hbm.at[pl.ds(base_row, CHUNK)])
        return carry

    lax.fori_loop(0, NCHUNK, chunk_body, 0)


_sc_fm = pl.kernel(
    _sc_body,
    out_type=jax.ShapeDtypeStruct((B, EMB), jnp.float32),
    mesh=plsc.VectorSubcoreMesh(core_axis_name="c", subcore_axis_name="s"),
    scratch_types=[
        pltpu.VMEM((GROUPS, IDX_PER_GROUP), jnp.int32),
        pltpu.VMEM((GROUPS, IDX_PER_GROUP, EMB), jnp.float32),
        pltpu.VMEM((CHUNK, EMB), jnp.float32),
        pltpu.SemaphoreType.DMA,
    ],
    compiler_params=pltpu.CompilerParams(use_tc_tiling_on_sc=False),
)

BS = 2048


def _dnn_body(fm_ref, wh0_ref, bh0_ref, wh1_ref, bh1_ref, wl_ref, bl_ref,
              bias_ref, out_ref):
    x = fm_ref[...]
    h = jax.nn.sigmoid(
        jnp.dot(x, wh0_ref[...], preferred_element_type=jnp.float32)
        + bh0_ref[...][None, :])
    h = jax.nn.sigmoid(
        jnp.dot(h, wh1_ref[...], preferred_element_type=jnp.float32)
        + bh1_ref[...][None, :])
    deep = jnp.sum(h * wl_ref[...][:, 0][None, :], axis=1)
    total = deep + jnp.sum(x, axis=1) + (bl_ref[...] + bias_ref[...])
    out_ref[...] = total[:, None]


_dnn = pl.pallas_call(
    _dnn_body,
    grid=(B // BS,),
    in_specs=[
        pl.BlockSpec((BS, EMB), lambda i: (i, 0)),
        pl.BlockSpec((EMB, H0), lambda i: (0, 0)),
        pl.BlockSpec((H0,), lambda i: (0,)),
        pl.BlockSpec((H0, H1), lambda i: (0, 0)),
        pl.BlockSpec((H1,), lambda i: (0,)),
        pl.BlockSpec((H1, 1), lambda i: (0, 0)),
        pl.BlockSpec((1,), lambda i: (0,)),
        pl.BlockSpec((1,), lambda i: (0,)),
    ],
    out_specs=pl.BlockSpec((BS, 1), lambda i: (i, 0)),
    out_shape=jax.ShapeDtypeStruct((B, 1), jnp.float32),
)


def kernel(cat_feat, flag, W1, W2, Wh0, bh0, Wh1, bh1, Wl, bl, bias):
    cat = cat_feat.astype(jnp.int32)
    field_offs = (jnp.arange(NUM_FIELDS, dtype=jnp.int32) * VOCAB)[None, :]
    flat_idx = (cat + field_offs).reshape(-1)
    w2_flat = W2.reshape(NUM_FIELDS * VOCAB, EMB)
    fm2 = _sc_fm(flat_idx, w2_flat)
    return _dnn(fm2, Wh0, bh0, Wh1, bh1, Wl, bl, bias)


# speedup vs baseline: 1.0793x; 1.0793x over previous
import functools

import jax
import jax.numpy as jnp
from jax import lax
from jax.experimental import pallas as pl
from jax.experimental.pallas import tpu as pltpu
from jax.experimental.pallas import tpu_sc as plsc

NUM_FIELDS = 26
VOCAB = 100000
EMB = 32
B = 16384
H0 = 256
H1 = 128

LANES = 16
NC = 2
NS = 16
NW = NC * NS
BPW = B // NW
RPG = 4
IDXPG = RPG * NUM_FIELDS
NGROUP = BPW // RPG
NBUF = 4
NBLK = NGROUP // NBUF


def _sc_body(flat_idx_hbm, w2_hbm, fm2_hbm, idx_all, rows_v, out_v,
             sem0, sem1, sem2, sem3):
    sems = (sem0, sem1, sem2, sem3)
    wid = lax.axis_index("c") * NS + lax.axis_index("s")
    pltpu.sync_copy(flat_idx_hbm.at[pl.ds(wid * NGROUP, NGROUP)], idx_all)

    def issue(g, b):
        pltpu.async_copy(w2_hbm.at[idx_all.at[g]], rows_v.at[b], sems[b])

    for b in range(NBUF):
        issue(b, b)

    def blk_body(blk, carry):
        for b in range(NBUF):
            g = blk * NBUF + b
            pltpu.make_async_copy(
                w2_hbm.at[idx_all.at[g]], rows_v.at[b], sems[b]).wait()
            obase = blk * (NBUF * RPG) + b * RPG
            for r in range(RPG):
                lo = r * NUM_FIELDS
                acc0 = jnp.zeros((LANES,), jnp.float32)
                acc1 = jnp.zeros((LANES,), jnp.float32)
                sq0 = jnp.zeros((LANES,), jnp.float32)
                sq1 = jnp.zeros((LANES,), jnp.float32)
                for j in range(NUM_FIELDS):
                    x0 = rows_v[b, lo + j, pl.ds(0, LANES)]
                    x1 = rows_v[b, lo + j, pl.ds(LANES, LANES)]
                    acc0 = acc0 + x0
                    sq0 = sq0 + x0 * x0
                    acc1 = acc1 + x1
                    sq1 = sq1 + x1 * x1
                out_v[obase + r, pl.ds(0, LANES)] = (acc0 * acc0 - sq0) * 0.005
                out_v[obase + r, pl.ds(LANES, LANES)] = (acc1 * acc1 - sq1) * 0.005
            @pl.when(g + NBUF < NGROUP)
            def _():
                issue(g + NBUF, b)
        return carry

    lax.fori_loop(0, NBLK, blk_body, 0)
    pltpu.sync_copy(out_v, fm2_hbm.at[pl.ds(wid * BPW, BPW)])


_sc_fm = pl.kernel(
    _sc_body,
    out_type=jax.ShapeDtypeStruct((B, EMB), jnp.float32),
    mesh=plsc.VectorSubcoreMesh(core_axis_name="c", subcore_axis_name="s"),
    scratch_types=[
        pltpu.VMEM((NGROUP, IDXPG), jnp.int32),
        pltpu.VMEM((NBUF, IDXPG, EMB), jnp.float32),
        pltpu.VMEM((BPW, EMB), jnp.float32),
        pltpu.SemaphoreType.DMA,
        pltpu.SemaphoreType.DMA,
        pltpu.SemaphoreType.DMA,
        pltpu.SemaphoreType.DMA,
    ],
    compiler_params=pltpu.CompilerParams(use_tc_tiling_on_sc=False),
)

BS = 2048


def _dnn_body(fm_ref, wh0_ref, bh0_ref, wh1_ref, bh1_ref, wl_ref, bl_ref,
              bias_ref, out_ref):
    x = fm_ref[...]
    h = jax.nn.sigmoid(
        jnp.dot(x, wh0_ref[...], preferred_element_type=jnp.float32)
        + bh0_ref[...][None, :])
    h = jax.nn.sigmoid(
        jnp.dot(h, wh1_ref[...], preferred_element_type=jnp.float32)
        + bh1_ref[...][None, :])
    deep = jnp.sum(h * wl_ref[...][:, 0][None, :], axis=1)
    total = deep + jnp.sum(x, axis=1) + (bl_ref[...] + bias_ref[...])
    out_ref[...] = total[:, None]


_dnn = pl.pallas_call(
    _dnn_body,
    grid=(B // BS,),
    in_specs=[
        pl.BlockSpec((BS, EMB), lambda i: (i, 0)),
        pl.BlockSpec((EMB, H0), lambda i: (0, 0)),
        pl.BlockSpec((H0,), lambda i: (0,)),
        pl.BlockSpec((H0, H1), lambda i: (0, 0)),
        pl.BlockSpec((H1,), lambda i: (0,)),
        pl.BlockSpec((H1, 1), lambda i: (0, 0)),
        pl.BlockSpec((1,), lambda i: (0,)),
        pl.BlockSpec((1,), lambda i: (0,)),
    ],
    out_specs=pl.BlockSpec((BS, 1), lambda i: (i, 0)),
    out_shape=jax.ShapeDtypeStruct((B, 1), jnp.float32),
)


def kernel(cat_feat, flag, W1, W2, Wh0, bh0, Wh1, bh1, Wl, bl, bias):
    cat = cat_feat.astype(jnp.int32)
    field_offs = (jnp.arange(NUM_FIELDS, dtype=jnp.int32) * VOCAB)[None, :]
    flat_idx = (cat + field_offs).reshape(B // RPG, IDXPG)
    w2_flat = W2.reshape(NUM_FIELDS * VOCAB, EMB)
    fm2 = _sc_fm(flat_idx, w2_flat)
    return _dnn(fm2, Wh0, bh0, Wh1, bh1, Wl, bl, bias)
